# baseline (speedup 1.0000x reference)
import jax
import jax.numpy as jnp
from jax import lax
from jax.experimental import pallas as pl
from jax.experimental.pallas import tpu as pltpu
from jax.experimental.pallas import tpu_sc as plsc

B, N, F, H = 256, 128, 9, 768
L = 16
K = F + 3
G = 4
NC, NS = 2, 16
NW = NC * NS
GPW = B // NW
NPW = GPW * N
STEPS = NPW // G
CPG = N // G
NP1 = N + 1


def _sc_body(idx_hbm, tab_hbm, gt_hbm, out_hbm,
             idx_v, rows_v, obuf_v, gt_v,
             sem_g0, sem_g1, sem_o0, sem_o1):
    wid = lax.axis_index("s") * NC + lax.axis_index("c")
    sem_g = (sem_g0, sem_g1)
    sem_o = (sem_o0, sem_o1)

    pltpu.sync_copy(gt_hbm, gt_v)
    pltpu.sync_copy(idx_hbm.at[pl.ds(wid * NPW * K, NPW * K)], idx_v)
    for gi in range(GPW):
        b = wid * GPW + gi
        pltpu.sync_copy(gt_v, out_hbm.at[pl.ds(b * NP1 * H, H)])

    def gather(s, slot):
        return pltpu.async_copy(
            tab_hbm.at[idx_v.at[pl.ds(s * G * K, G * K)]],
            rows_v.at[slot], sem_g[slot])

    def out_slice(s):
        orow = (wid * GPW + s // CPG) * NP1 + 1 + (s % CPG) * G
        return out_hbm.at[pl.ds(orow * H, G * H)]

    def compute(slot):
        def col_body(j, carry):
            off = j * L
            for g in range(G):
                r = g * K
                acc = rows_v[slot, r, pl.ds(off, L)]
                for f in range(1, K):
                    acc = acc + rows_v[slot, r + f, pl.ds(off, L)]
                obuf_v[slot, pl.ds(g * H + off, L)] = acc
            return carry
        lax.fori_loop(0, H // L, col_body, 0)

    gather(0, 0).wait()

    def pair_body(p, carry):
        s0 = 2 * p
        c1 = gather(s0 + 1, 1)

        @pl.when(p > 0)
        def _():
            pltpu.make_async_copy(obuf_v.at[0], out_slice(s0), sem_o0).wait()
        compute(0)
        pltpu.async_copy(obuf_v.at[0], out_slice(s0), sem_o0)

        @pl.when(s0 + 2 < STEPS)
        def _():
            gather(s0 + 2, 0)
        c1.wait()

        @pl.when(p > 0)
        def _():
            pltpu.make_async_copy(obuf_v.at[1], out_slice(s0 + 1), sem_o1).wait()
        compute(1)
        pltpu.async_copy(obuf_v.at[1], out_slice(s0 + 1), sem_o1)

        @pl.when(s0 + 2 < STEPS)
        def _():
            pltpu.make_async_copy(
                tab_hbm.at[idx_v.at[pl.ds(0, G * K)]],
                rows_v.at[0], sem_g0).wait()
        return carry

    lax.fori_loop(0, STEPS // 2, pair_body, 0)
    pltpu.make_async_copy(obuf_v.at[0], out_slice(STEPS - 2), sem_o0).wait()
    pltpu.make_async_copy(obuf_v.at[1], out_slice(STEPS - 1), sem_o1).wait()


def kernel(x, in_degree, out_degree, atom_table, in_degree_table,
           out_degree_table, graph_token, token_init):
    del token_init
    na = atom_table.shape[0]
    ni = in_degree_table.shape[0]
    tab = jnp.concatenate([atom_table, in_degree_table, out_degree_table],
                          axis=0)
    idx = jnp.concatenate(
        [x,
         in_degree[..., None] + na,
         out_degree[..., None] + (na + ni),
         jnp.zeros((B, N, 1), jnp.int32)],
        axis=-1).reshape(-1)
    gt_flat = graph_token.reshape(-1)
    mesh = plsc.VectorSubcoreMesh(core_axis_name="c", subcore_axis_name="s")
    k = pl.kernel(
        _sc_body,
        out_type=jax.ShapeDtypeStruct((B * NP1 * H,), jnp.float32),
        mesh=mesh,
        scratch_types=[
            pltpu.VMEM((NPW * K,), jnp.int32),
            pltpu.VMEM((2, G * K, H), jnp.float32),
            pltpu.VMEM((2, G * H), jnp.float32),
            pltpu.VMEM((H,), jnp.float32),
            pltpu.SemaphoreType.DMA,
            pltpu.SemaphoreType.DMA,
            pltpu.SemaphoreType.DMA,
            pltpu.SemaphoreType.DMA,
        ],
    )
    out_flat = k(idx, tab, gt_flat)
    return out_flat.reshape(B, NP1, H)


# speedup vs baseline: 3.0268x; 3.0268x over previous
import jax
import jax.numpy as jnp
from jax import lax
from jax.experimental import pallas as pl
from jax.experimental.pallas import tpu as pltpu
from jax.experimental.pallas import tpu_sc as plsc

B, N, F, H = 256, 128, 9, 768
L = 16
K = F + 2
G = 8
NC, NS = 2, 16
NW = NC * NS
GPW = B // NW
NPW = GPW * N
STEPS = NPW // G
CPG = N // G
NP1 = N + 1
W2 = H // 2
JW = W2 // L
HIMASK = -65536


def _sc_body(idx_hbm, tab_hbm, gt_hbm, out_hbm,
             idx_v, rows_v, obuf_v, gt_v,
             sem_g0, sem_g1, sem_o0, sem_o1):
    wid = lax.axis_index("s") * NC + lax.axis_index("c")
    sem_g = (sem_g0, sem_g1)

    pltpu.sync_copy(gt_hbm, gt_v)
    pltpu.sync_copy(idx_hbm.at[pl.ds(wid * NPW * K, NPW * K)], idx_v)
    for gi in range(GPW):
        b = wid * GPW + gi
        pltpu.sync_copy(gt_v, out_hbm.at[pl.ds(b * NP1 * H, H)])

    def gather(s, slot):
        return pltpu.async_copy(
            tab_hbm.at[idx_v.at[pl.ds(s * G * K, G * K)]],
            rows_v.at[slot], sem_g[slot])

    def out_slice(s):
        orow = (wid * GPW + s // CPG) * NP1 + 1 + (s % CPG) * G
        return out_hbm.at[pl.ds(orow * H, G * H)]

    def f32view(v):
        return lax.bitcast_convert_type(v, jnp.float32)

    def compute(slot):
        def col_body(j, carry):
            off = j * L
            for g in range(G):
                r = g * K
                w = rows_v[slot, r, pl.ds(off, L)]
                acc_lo = f32view(w << 16)
                acc_hi = f32view(w & HIMASK)
                for f in range(1, K):
                    w = rows_v[slot, r + f, pl.ds(off, L)]
                    acc_lo = acc_lo + f32view(w << 16)
                    acc_hi = acc_hi + f32view(w & HIMASK)
                obuf_v[slot, pl.ds(g * H + off, L)] = acc_lo
                obuf_v[slot, pl.ds(g * H + W2 + off, L)] = acc_hi
            return carry
        lax.fori_loop(0, JW, col_body, 0)

    gather(0, 0).wait()

    def pair_body(p, carry):
        s0 = 2 * p
        c1 = gather(s0 + 1, 1)

        @pl.when(p > 0)
        def _():
            pltpu.make_async_copy(obuf_v.at[0], out_slice(s0), sem_o0).wait()
        compute(0)
        pltpu.async_copy(obuf_v.at[0], out_slice(s0), sem_o0)

        @pl.when(s0 + 2 < STEPS)
        def _():
            gather(s0 + 2, 0)
        c1.wait()

        @pl.when(p > 0)
        def _():
            pltpu.make_async_copy(obuf_v.at[1], out_slice(s0 + 1), sem_o1).wait()
        compute(1)
        pltpu.async_copy(obuf_v.at[1], out_slice(s0 + 1), sem_o1)

        @pl.when(s0 + 2 < STEPS)
        def _():
            pltpu.make_async_copy(
                tab_hbm.at[idx_v.at[pl.ds(0, G * K)]],
                rows_v.at[0], sem_g0).wait()
        return carry

    lax.fori_loop(0, STEPS // 2, pair_body, 0)
    pltpu.make_async_copy(obuf_v.at[0], out_slice(STEPS - 2), sem_o0).wait()
    pltpu.make_async_copy(obuf_v.at[1], out_slice(STEPS - 1), sem_o1).wait()


def kernel(x, in_degree, out_degree, atom_table, in_degree_table,
           out_degree_table, graph_token, token_init):
    del token_init
    na = atom_table.shape[0]
    ni = in_degree_table.shape[0]

    tab = jnp.concatenate([atom_table, in_degree_table, out_degree_table],
                          axis=0).astype(jnp.bfloat16)
    pairs = jnp.stack([tab[:, :W2], tab[:, W2:]], axis=-1)
    tab_pk = lax.bitcast_convert_type(pairs, jnp.int32)

    idx = jnp.concatenate(
        [x,
         in_degree[..., None] + na,
         out_degree[..., None] + (na + ni)],
        axis=-1).reshape(-1)
    gt_flat = graph_token.reshape(-1)
    mesh = plsc.VectorSubcoreMesh(core_axis_name="c", subcore_axis_name="s")
    k = pl.kernel(
        _sc_body,
        out_type=jax.ShapeDtypeStruct((B * NP1 * H,), jnp.float32),
        mesh=mesh,
        scratch_types=[
            pltpu.VMEM((NPW * K,), jnp.int32),
            pltpu.VMEM((2, G * K, W2), jnp.int32),
            pltpu.VMEM((2, G * H), jnp.float32),
            pltpu.VMEM((H,), jnp.float32),
            pltpu.SemaphoreType.DMA,
            pltpu.SemaphoreType.DMA,
            pltpu.SemaphoreType.DMA,
            pltpu.SemaphoreType.DMA,
        ],
    )
    out_flat = k(idx, tab_pk, gt_flat)
    return out_flat.reshape(B, NP1, H)


# speedup vs baseline: 3.1442x; 1.0388x over previous
import jax
import jax.numpy as jnp
from jax import lax
from jax.experimental import pallas as pl
from jax.experimental.pallas import tpu as pltpu
from jax.experimental.pallas import tpu_sc as plsc

B, N, F, H = 256, 128, 9, 768
L = 16
K = F + 2
G = 8
NC, NS = 2, 16
NW = NC * NS
GPW = B // NW
NPW = GPW * N
STEPS = NPW // G
CPG = N // G
NP1 = N + 1
W2 = H // 2
JW = W2 // L
HIMASK = -65536


def _sc_body(idx_hbm, tab_hbm, gt_hbm, out_hbm,
             idx_v, rows_v, obuf_v, gt_v,
             sem_g0, sem_g1, sem_o0, sem_o1):
    wid = lax.axis_index("s") * NC + lax.axis_index("c")
    sem_g = (sem_g0, sem_g1)

    pltpu.sync_copy(gt_hbm, gt_v)
    pltpu.sync_copy(idx_hbm.at[pl.ds(wid * NPW * K, NPW * K)], idx_v)
    for gi in range(GPW):
        b = wid * GPW + gi
        pltpu.sync_copy(gt_v, out_hbm.at[pl.ds(b * NP1 * H, H)])

    def gather(s, slot):
        return pltpu.async_copy(
            tab_hbm.at[idx_v.at[pl.ds(s * G * K, G * K)]],
            rows_v.at[slot], sem_g[slot])

    def out_slice(s):
        orow = (wid * GPW + s // CPG) * NP1 + 1 + (s % CPG) * G
        return out_hbm.at[pl.ds(orow * H, G * H)]

    def f32view(v):
        return lax.bitcast_convert_type(v, jnp.float32)

    def compute(slot):
        def col_body(j, carry):
            off = j * L
            for g in range(G):
                r = g * K
                w = rows_v[slot, r, pl.ds(off, L)]
                acc_lo = f32view(w << 16)
                acc_hi = f32view(w)
                for f in range(1, K):
                    w = rows_v[slot, r + f, pl.ds(off, L)]
                    acc_lo = acc_lo + f32view(w << 16)
                    acc_hi = acc_hi + f32view(w)
                obuf_v[slot, pl.ds(g * H + off, L)] = acc_lo
                obuf_v[slot, pl.ds(g * H + W2 + off, L)] = acc_hi
            return carry
        lax.fori_loop(0, JW, col_body, 0)

    gather(0, 0).wait()

    def pair_body(p, carry):
        s0 = 2 * p
        c1 = gather(s0 + 1, 1)

        @pl.when(p > 0)
        def _():
            pltpu.make_async_copy(obuf_v.at[0], out_slice(s0), sem_o0).wait()
        compute(0)
        pltpu.async_copy(obuf_v.at[0], out_slice(s0), sem_o0)

        @pl.when(s0 + 2 < STEPS)
        def _():
            gather(s0 + 2, 0)
        c1.wait()

        @pl.when(p > 0)
        def _():
            pltpu.make_async_copy(obuf_v.at[1], out_slice(s0 + 1), sem_o1).wait()
        compute(1)
        pltpu.async_copy(obuf_v.at[1], out_slice(s0 + 1), sem_o1)

        @pl.when(s0 + 2 < STEPS)
        def _():
            pltpu.make_async_copy(
                tab_hbm.at[idx_v.at[pl.ds(0, G * K)]],
                rows_v.at[0], sem_g0).wait()
        return carry

    lax.fori_loop(0, STEPS // 2, pair_body, 0)
    pltpu.make_async_copy(obuf_v.at[0], out_slice(STEPS - 2), sem_o0).wait()
    pltpu.make_async_copy(obuf_v.at[1], out_slice(STEPS - 1), sem_o1).wait()


def kernel(x, in_degree, out_degree, atom_table, in_degree_table,
           out_degree_table, graph_token, token_init):
    del token_init
    na = atom_table.shape[0]
    ni = in_degree_table.shape[0]

    tab = jnp.concatenate([atom_table, in_degree_table, out_degree_table],
                          axis=0)
    lo = lax.bitcast_convert_type(tab[:, :W2].astype(jnp.bfloat16),
                                  jnp.uint16).astype(jnp.int32)
    hi = lax.bitcast_convert_type(
        (tab[:, W2:] * (1.0 / (1.0 + 2.0 ** -8))).astype(jnp.bfloat16),
        jnp.uint16).astype(jnp.int32)
    tab_pk = lo | (hi << 16)

    idx = jnp.concatenate(
        [x,
         in_degree[..., None] + na,
         out_degree[..., None] + (na + ni)],
        axis=-1).reshape(-1)
    gt_flat = graph_token.reshape(-1)
    mesh = plsc.VectorSubcoreMesh(core_axis_name="c", subcore_axis_name="s")
    k = pl.kernel(
        _sc_body,
        out_type=jax.ShapeDtypeStruct((B * NP1 * H,), jnp.float32),
        mesh=mesh,
        scratch_types=[
            pltpu.VMEM((NPW * K,), jnp.int32),
            pltpu.VMEM((2, G * K, W2), jnp.int32),
            pltpu.VMEM((2, G * H), jnp.float32),
            pltpu.VMEM((H,), jnp.float32),
            pltpu.SemaphoreType.DMA,
            pltpu.SemaphoreType.DMA,
            pltpu.SemaphoreType.DMA,
            pltpu.SemaphoreType.DMA,
        ],
    )
    out_flat = k(idx, tab_pk, gt_flat)
    return out_flat.reshape(B, NP1, H)


# speedup vs baseline: 3.1488x; 1.0015x over previous
import jax
import jax.numpy as jnp
from jax import lax
from jax.experimental import pallas as pl
from jax.experimental.pallas import tpu as pltpu
from jax.experimental.pallas import tpu_sc as plsc

B, N, F, H = 256, 128, 9, 768
L = 16
K = F + 2
G = 8
NC, NS = 2, 16
NW = NC * NS
GPW = B // NW
NPW = GPW * N
STEPS = NPW // G
CPG = N // G
NP1 = N + 1
W2 = H // 2
JW = W2 // L
HIMASK = -65536


def _sc_body(idx_hbm, tab_hbm, gt_hbm, out_hbm,
             idx_v, rows_v, obuf_v, gt_v,
             sem_g0, sem_g1, sem_o0, sem_o1, sem_h0, sem_h1):
    wid = lax.axis_index("s") * NC + lax.axis_index("c")
    sem_g = (sem_g0, sem_g1)
    sem_h = (sem_h0, sem_h1)
    HALF = 48
    REST = G * K - HALF

    pltpu.sync_copy(gt_hbm, gt_v)
    pltpu.sync_copy(idx_hbm.at[pl.ds(wid * NPW * K, NPW * K)], idx_v)
    for gi in range(GPW):
        b = wid * GPW + gi
        pltpu.sync_copy(gt_v, out_hbm.at[pl.ds(b * NP1 * H, H)])

    def gather(s, slot):
        pltpu.async_copy(
            tab_hbm.at[idx_v.at[pl.ds(s * G * K, HALF)]],
            rows_v.at[slot, pl.ds(0, HALF)], sem_g[slot])
        pltpu.async_copy(
            tab_hbm.at[idx_v.at[pl.ds(s * G * K + HALF, REST)]],
            rows_v.at[slot, pl.ds(HALF, REST)], sem_h[slot])

    def gwait(slot):
        pltpu.make_async_copy(
            tab_hbm.at[idx_v.at[pl.ds(0, HALF)]],
            rows_v.at[slot, pl.ds(0, HALF)], sem_g[slot]).wait()
        pltpu.make_async_copy(
            tab_hbm.at[idx_v.at[pl.ds(0, REST)]],
            rows_v.at[slot, pl.ds(HALF, REST)], sem_h[slot]).wait()

    def out_slice(s):
        orow = (wid * GPW + s // CPG) * NP1 + 1 + (s % CPG) * G
        return out_hbm.at[pl.ds(orow * H, G * H)]

    def f32view(v):
        return lax.bitcast_convert_type(v, jnp.float32)

    def compute(slot):
        def col_body(j, carry):
            off = j * L
            for g in range(G):
                r = g * K
                w = rows_v[slot, r, pl.ds(off, L)]
                acc_lo = f32view(w << 16)
                acc_hi = f32view(w)
                for f in range(1, K):
                    w = rows_v[slot, r + f, pl.ds(off, L)]
                    acc_lo = acc_lo + f32view(w << 16)
                    acc_hi = acc_hi + f32view(w)
                obuf_v[slot, pl.ds(g * H + off, L)] = acc_lo
                obuf_v[slot, pl.ds(g * H + W2 + off, L)] = acc_hi
            return carry
        lax.fori_loop(0, JW, col_body, 0)

    gather(0, 0)
    gwait(0)

    def pair_body(p, carry):
        s0 = 2 * p
        gather(s0 + 1, 1)

        @pl.when(p > 0)
        def _():
            pltpu.make_async_copy(obuf_v.at[0], out_slice(s0), sem_o0).wait()
        compute(0)
        pltpu.async_copy(obuf_v.at[0], out_slice(s0), sem_o0)

        @pl.when(s0 + 2 < STEPS)
        def _():
            gather(s0 + 2, 0)
        gwait(1)

        @pl.when(p > 0)
        def _():
            pltpu.make_async_copy(obuf_v.at[1], out_slice(s0 + 1), sem_o1).wait()
        compute(1)
        pltpu.async_copy(obuf_v.at[1], out_slice(s0 + 1), sem_o1)

        @pl.when(s0 + 2 < STEPS)
        def _():
            gwait(0)
        return carry

    lax.fori_loop(0, STEPS // 2, pair_body, 0)
    pltpu.make_async_copy(obuf_v.at[0], out_slice(STEPS - 2), sem_o0).wait()
    pltpu.make_async_copy(obuf_v.at[1], out_slice(STEPS - 1), sem_o1).wait()


def kernel(x, in_degree, out_degree, atom_table, in_degree_table,
           out_degree_table, graph_token, token_init):
    del token_init
    na = atom_table.shape[0]
    ni = in_degree_table.shape[0]

    tab = jnp.concatenate([atom_table, in_degree_table, out_degree_table],
                          axis=0)
    lo = lax.bitcast_convert_type(tab[:, :W2].astype(jnp.bfloat16),
                                  jnp.uint16).astype(jnp.int32)
    hi = lax.bitcast_convert_type(
        (tab[:, W2:] * (1.0 / (1.0 + 2.0 ** -8))).astype(jnp.bfloat16),
        jnp.uint16).astype(jnp.int32)
    tab_pk = lo | (hi << 16)

    idx = jnp.concatenate(
        [x,
         in_degree[..., None] + na,
         out_degree[..., None] + (na + ni)],
        axis=-1).reshape(-1)
    gt_flat = graph_token.reshape(-1)
    mesh = plsc.VectorSubcoreMesh(core_axis_name="c", subcore_axis_name="s")
    k = pl.kernel(
        _sc_body,
        out_type=jax.ShapeDtypeStruct((B * NP1 * H,), jnp.float32),
        mesh=mesh,
        scratch_types=[
            pltpu.VMEM((NPW * K,), jnp.int32),
            pltpu.VMEM((2, G * K, W2), jnp.int32),
            pltpu.VMEM((2, G * H), jnp.float32),
            pltpu.VMEM((H,), jnp.float32),
            pltpu.SemaphoreType.DMA,
            pltpu.SemaphoreType.DMA,
            pltpu.SemaphoreType.DMA,
            pltpu.SemaphoreType.DMA,
            pltpu.SemaphoreType.DMA,
            pltpu.SemaphoreType.DMA,
        ],
    )
    out_flat = k(idx, tab_pk, gt_flat)
    return out_flat.reshape(B, NP1, H)
